# Initial kernel scaffold; baseline (speedup 1.0000x reference)
import jax
import jax.numpy as jnp
from jax.experimental import pallas as pl


def kernel(input_ids, emb_table):
    raise NotImplementedError("write your pallas kernel here")


# speedup vs baseline: 4.6028x; 4.6028x over previous
import functools

import jax
import jax.numpy as jnp
from jax import lax
from jax.experimental import pallas as pl
from jax.experimental.pallas import tpu as pltpu
from jax.experimental.pallas import tpu_sc as plsc

VOCAB = 100000
D_MODEL = 128
MAX_LEN = 512
BATCH = 4096
SEQ = 200

NUM_CORES = 2
NUM_SUBCORES = 16
NW = NUM_CORES * NUM_SUBCORES
SEQ_PER_W = BATCH // NW
LANES = 16
G0, G1 = 104, 96


def _sine_pe():
    pos = jnp.arange(MAX_LEN, dtype=jnp.float32)[:, None]
    div = jnp.exp(
        jnp.arange(0, D_MODEL, 2, dtype=jnp.float32)
        * (-jnp.log(10000.0) / D_MODEL)
    )
    pe = jnp.zeros((MAX_LEN, D_MODEL), dtype=jnp.float32)
    pe = pe.at[:, 0::2].set(jnp.sin(pos * div))
    pe = pe.at[:, 1::2].set(jnp.cos(pos * div))
    return pe[:SEQ]


def _body(ids, table, pe, out, idx_v, pe_v, rows, gsem):
    c = lax.axis_index("c")
    s = lax.axis_index("s")
    wid = s * NUM_CORES + c
    base = wid * SEQ_PER_W

    pltpu.sync_copy(pe, pe_v)
    pltpu.sync_copy(ids.at[pl.ds(base, SEQ_PER_W)], idx_v)

    def seq_body(g, carry):
        c1 = pltpu.async_copy(
            table.at[idx_v.at[g, pl.ds(0, G0)]], rows.at[pl.ds(0, G0)], gsem
        )
        c2 = pltpu.async_copy(
            table.at[idx_v.at[g, pl.ds(G0, G1)]], rows.at[pl.ds(G0, G1)], gsem
        )
        c1.wait()
        c2.wait()

        def add_body(r, carry2):
            for cc in range(D_MODEL // LANES):
                plsc.addupdate(
                    rows.at[r, pl.ds(cc * LANES, LANES)],
                    pe_v[r, pl.ds(cc * LANES, LANES)],
                )
            return carry2

        lax.fori_loop(0, SEQ, add_body, 0, unroll=2)
        pltpu.sync_copy(rows, out.at[base + g])
        return carry

    lax.fori_loop(0, SEQ_PER_W, seq_body, 0)


def kernel(input_ids, emb_table):
    batch, seq = input_ids.shape
    _, d = emb_table.shape
    pe = _sine_pe()
    fn = pl.kernel(
        _body,
        out_type=jax.ShapeDtypeStruct((batch, seq, d), jnp.float32),
        mesh=plsc.VectorSubcoreMesh(
            core_axis_name="c", subcore_axis_name="s"
        ),
        compiler_params=pltpu.CompilerParams(use_tc_tiling_on_sc=False),
        scratch_types=[
            pltpu.VMEM((SEQ_PER_W, SEQ), jnp.int32),
            pltpu.VMEM((SEQ, D_MODEL), jnp.float32),
            pltpu.VMEM((SEQ, D_MODEL), jnp.float32),
            pltpu.SemaphoreType.DMA,
        ],
    )
    return fn(input_ids.astype(jnp.int32), emb_table, pe)


# speedup vs baseline: 7.6505x; 1.6622x over previous
import jax
import jax.numpy as jnp
from jax import lax
from jax.experimental import pallas as pl
from jax.experimental.pallas import tpu as pltpu
from jax.experimental.pallas import tpu_sc as plsc

VOCAB = 100000
D_MODEL = 128
MAX_LEN = 512
BATCH = 4096
SEQ = 200

NUM_CORES = 2
NUM_SUBCORES = 16
NW = NUM_CORES * NUM_SUBCORES
ROWS_PER_W = BATCH * SEQ // NW
LANES = 16
SLOTS = ((0, 104, 0), (104, 96, 104), (200, 104, 0), (304, 96, 104))
NBUF = 4
NP = ROWS_PER_W // 400


def _sine_pe():
    pos = jnp.arange(MAX_LEN, dtype=jnp.float32)[:, None]
    div = jnp.exp(
        jnp.arange(0, D_MODEL, 2, dtype=jnp.float32)
        * (-jnp.log(10000.0) / D_MODEL)
    )
    pe = jnp.zeros((MAX_LEN, D_MODEL), dtype=jnp.float32)
    pe = pe.at[:, 0::2].set(jnp.sin(pos * div))
    pe = pe.at[:, 1::2].set(jnp.cos(pos * div))
    return pe[:SEQ]


def _body(ids, table, pe, out, idx_v, pe_v, r0, r1, r2, r3,
          g0, g1, g2, g3, o0, o1, o2, o3):
    c = lax.axis_index("c")
    s = lax.axis_index("s")
    wid = s * NUM_CORES + c
    base = wid * ROWS_PER_W

    rows = (r0, r1, r2, r3)
    gs = (g0, g1, g2, g3)
    os_ = (o0, o1, o2, o3)

    pltpu.sync_copy(pe, pe_v)
    pltpu.sync_copy(ids.at[pl.ds(base, ROWS_PER_W)], idx_v)

    def gather_copy(p, b):
        off, ln, _ = SLOTS[b]
        lo = p * 400 + off
        return pltpu.make_async_copy(
            table.at[idx_v.at[pl.ds(lo, ln)]],
            rows[b].at[pl.ds(0, ln)],
            gs[b],
        )

    def out_copy(p, b):
        off, ln, _ = SLOTS[b]
        lo = p * 400 + off
        return pltpu.make_async_copy(
            rows[b].at[pl.ds(0, ln)],
            out.at[pl.ds(base + lo, ln)],
            os_[b],
        )

    def add_pe(p, b):
        _, ln, pe_off = SLOTS[b]

        def row_body(r, carry):
            for cc in range(D_MODEL // LANES):
                plsc.addupdate(
                    rows[b].at[r, pl.ds(cc * LANES, LANES)],
                    pe_v[pe_off + r, pl.ds(cc * LANES, LANES)],
                )
            return carry

        lax.fori_loop(0, ln, row_body, 0, unroll=2)

    gather_copy(0, 0).start()

    def pipe_body(p, carry):
        for b in range(NBUF):
            gather_copy(p, b).wait()
            if b < NBUF - 1:
                @pl.when(p >= 1)
                def _():
                    out_copy(p - 1, b + 1).wait()

                gather_copy(p, b + 1).start()
            else:
                @pl.when(p + 1 < NP)
                def _():
                    out_copy(p, 0).wait()
                    gather_copy(p + 1, 0).start()

            add_pe(p, b)
            out_copy(p, b).start()
        return carry

    lax.fori_loop(0, NP, pipe_body, 0)
    for b in range(NBUF):
        out_copy(NP - 1, b).wait()


def kernel(input_ids, emb_table):
    batch, seq = input_ids.shape
    _, d = emb_table.shape
    pe = _sine_pe()
    fn = pl.kernel(
        _body,
        out_type=jax.ShapeDtypeStruct((batch * seq, d), jnp.float32),
        mesh=plsc.VectorSubcoreMesh(
            core_axis_name="c", subcore_axis_name="s"
        ),
        compiler_params=pltpu.CompilerParams(use_tc_tiling_on_sc=False),
        scratch_types=(
            [pltpu.VMEM((ROWS_PER_W,), jnp.int32)]
            + [pltpu.VMEM((SEQ, D_MODEL), jnp.float32)]
            + [pltpu.VMEM((104, D_MODEL), jnp.float32)] * NBUF
            + [pltpu.SemaphoreType.DMA] * (2 * NBUF)
        ),
    )
    flat = fn(input_ids.reshape(-1).astype(jnp.int32), emb_table, pe)
    return flat.reshape(batch, seq, d)
